# baseline (speedup 1.0000x reference)
import jax
import jax.numpy as jnp
from jax.experimental import pallas as pl
from jax.experimental.pallas import tpu as pltpu

H0 = W0 = 32
KS = 5
CP = 128
G = 8
SX = 1040
S1R = 520
S2R = 260
NOUT = 10


def _lenet_kernel(x_ref, w1_ref, b1_ref, w2_ref, b2_ref,
                  wf1_ref, bf1_ref, wf2_ref, bf2_ref, wf3_ref, bf3_ref,
                  o_ref):
    f32 = jnp.float32
    xf = x_ref[...].reshape(G * SX, 8)

    L = G * SX - 8
    xs = jnp.concatenate([xf[kw:kw + L] for kw in range(KS)], axis=1)
    L2 = G * SX - 136
    acc = jnp.dot(xs[0:L2], w1_ref[0], preferred_element_type=f32)
    for kh in range(1, KS):
        acc = acc + jnp.dot(xs[32 * kh:32 * kh + L2], w1_ref[kh],
                            preferred_element_type=f32)
    r1 = jnp.maximum(acc + b1_ref[...], 0.0)

    Q = G * S1R - 88
    e = r1[0:2 * Q].reshape(Q, 2, CP)
    f = r1[32:32 + 2 * Q].reshape(Q, 2, CP)
    m1 = jnp.maximum(jnp.maximum(e[:, 0], e[:, 1]),
                     jnp.maximum(f[:, 0], f[:, 1]))

    p1f = jnp.concatenate([m1[:, :8], jnp.zeros((88, 8), f32)], axis=0)
    L3 = G * S1R - 136
    s2 = jnp.concatenate(
        [p1f[32 * kh + kw:32 * kh + kw + L3] for kh in range(KS) for kw in range(KS)],
        axis=1)
    acc2 = jnp.dot(s2, w2_ref[...], preferred_element_type=f32)
    r2 = jnp.maximum(acc2 + b2_ref[...], 0.0)

    V = G * S2R - 120
    e2 = r2[0:2 * V].reshape(V, 2, CP)
    f2 = r2[32:32 + 2 * V].reshape(V, 2, CP)
    m2 = jnp.maximum(jnp.maximum(e2[:, 0], e2[:, 1]),
                     jnp.maximum(f2[:, 0], f2[:, 1]))

    m2p = jnp.concatenate([m2, jnp.zeros((120, CP), f32)], axis=0)
    grid5 = m2p.reshape(G, S2R, CP)[:, :160].reshape(G, 5, 32, CP)[:, :, :5]
    feats = grid5.reshape(G, 25, CP).reshape(G, 25 * CP)

    h = jnp.maximum(jnp.dot(feats, wf1_ref[...],
                            preferred_element_type=f32) + bf1_ref[...], 0.0)
    h = jnp.maximum(jnp.dot(h, wf2_ref[...],
                            preferred_element_type=f32) + bf2_ref[...], 0.0)
    o_ref[...] = (jnp.dot(h, wf3_ref[...],
                          preferred_element_type=f32) + bf3_ref[...]).astype(o_ref.dtype)


@jax.jit
def _forward(x_nchw, w1k, b1, w2k, b2, w_fc1, b_fc1, w_fc2, b_fc2, w_fc3, b_fc3):
    B = x_nchw.shape[0]
    x = jnp.transpose(x_nchw, (0, 2, 3, 1)).reshape(B, H0 * W0, 3)
    x = jnp.pad(x, ((0, 0), (0, SX - H0 * W0), (0, 5)))
    w2c = w2k[:, :8, :].reshape(200, CP)
    wf1 = w_fc1.reshape(32, CP, CP)[:25].reshape(25 * CP, CP)

    out = pl.pallas_call(
        _lenet_kernel,
        out_shape=jax.ShapeDtypeStruct((B, CP), jnp.float32),
        grid=(B // G,),
        in_specs=[
            pl.BlockSpec((G, SX, 8), lambda i: (i, 0, 0)),
            pl.BlockSpec((KS, 40, CP), lambda i: (0, 0, 0)),
            pl.BlockSpec((1, CP), lambda i: (0, 0)),
            pl.BlockSpec((200, CP), lambda i: (0, 0)),
            pl.BlockSpec((1, CP), lambda i: (0, 0)),
            pl.BlockSpec((25 * CP, CP), lambda i: (0, 0)),
            pl.BlockSpec((1, CP), lambda i: (0, 0)),
            pl.BlockSpec((CP, CP), lambda i: (0, 0)),
            pl.BlockSpec((1, CP), lambda i: (0, 0)),
            pl.BlockSpec((CP, CP), lambda i: (0, 0)),
            pl.BlockSpec((1, CP), lambda i: (0, 0)),
        ],
        out_specs=pl.BlockSpec((G, CP), lambda i: (i, 0)),
        compiler_params=pltpu.CompilerParams(
            dimension_semantics=("parallel",),
            vmem_limit_bytes=64 * 1024 * 1024),
    )(x, w1k, b1, w2c, b2, wf1, b_fc1, w_fc2, b_fc2, w_fc3, b_fc3)
    return out[:, :NOUT]


def kernel(x_nchw, w1k, b1, w2k, b2, S1, S2,
           w_fc1, b_fc1, w_fc2, b_fc2, w_fc3, b_fc3):
    del S1, S2
    return _forward(x_nchw, w1k, b1, w2k, b2,
                    w_fc1, b_fc1, w_fc2, b_fc2, w_fc3, b_fc3)


# speedup vs baseline: 1.1840x; 1.1840x over previous
import jax
import jax.numpy as jnp
from jax.experimental import pallas as pl
from jax.experimental.pallas import tpu as pltpu

H0 = W0 = 32
KS = 5
CP = 128
G = 8
SX = 1040
S1R = 520
S2R = 260
FM = 512
NOUT = 10


def _conv_kernel(x_ref, w1_ref, b1_ref, w2_ref, b2_ref, o_ref):
    f32 = jnp.float32
    xf = x_ref[...].reshape(G * SX, 8)

    L = G * SX - 8
    xs = jnp.concatenate([xf[kw:kw + L] for kw in range(KS)], axis=1)
    L2 = G * SX - 136
    acc = jnp.dot(xs[0:L2], w1_ref[0], preferred_element_type=f32)
    for kh in range(1, KS):
        acc = acc + jnp.dot(xs[32 * kh:32 * kh + L2], w1_ref[kh],
                            preferred_element_type=f32)
    r1 = jnp.maximum(acc + b1_ref[...], 0.0)

    Q = G * S1R - 88
    e = r1[0:2 * Q].reshape(Q, 2, CP)
    f = r1[32:32 + 2 * Q].reshape(Q, 2, CP)
    m1 = jnp.maximum(jnp.maximum(e[:, 0], e[:, 1]),
                     jnp.maximum(f[:, 0], f[:, 1]))

    p1f = jnp.concatenate([m1[:, :8], jnp.zeros((88, 8), f32)], axis=0)
    L3 = G * S1R - 136
    L3s = G * S1R - 8
    p1s = jnp.concatenate([p1f[kw:kw + L3s] for kw in range(KS)], axis=1)
    acc2 = jnp.dot(p1s[0:L3], w2_ref[0], preferred_element_type=f32)
    for kh in range(1, KS):
        acc2 = acc2 + jnp.dot(p1s[32 * kh:32 * kh + L3], w2_ref[kh],
                              preferred_element_type=f32)
    r2 = jnp.maximum(acc2 + b2_ref[...], 0.0)

    V = G * S2R - 120
    e2 = r2[0:2 * V].reshape(V, 2, CP)
    f2 = r2[32:32 + 2 * V].reshape(V, 2, CP)
    m2 = jnp.maximum(jnp.maximum(e2[:, 0], e2[:, 1]),
                     jnp.maximum(f2[:, 0], f2[:, 1]))

    m2p = jnp.concatenate([m2, jnp.zeros((120, CP), jnp.float32)], axis=0)
    grid5 = m2p.reshape(G, S2R, CP)[:, :160].reshape(G, 5, 32, CP)[:, :, :5]
    o_ref[...] = grid5.reshape(G, 25, CP).reshape(G, 25 * CP)


def _fc_kernel(x_ref, w1_ref, b1_ref, w2_ref, b2_ref, w3_ref, b3_ref, o_ref):
    f32 = jnp.float32
    h = jnp.maximum(jnp.dot(x_ref[...], w1_ref[...],
                            preferred_element_type=f32) + b1_ref[...], 0.0)
    h = jnp.maximum(jnp.dot(h, w2_ref[...],
                            preferred_element_type=f32) + b2_ref[...], 0.0)
    o_ref[...] = jnp.dot(h, w3_ref[...],
                         preferred_element_type=f32) + b3_ref[...]


@jax.jit
def _forward(x_nchw, w1k, b1, w2k, b2, w_fc1, b_fc1, w_fc2, b_fc2, w_fc3, b_fc3):
    B = x_nchw.shape[0]
    x = jnp.transpose(x_nchw, (0, 2, 3, 1)).reshape(B, H0 * W0, 3)
    x = jnp.pad(x, ((0, 0), (0, SX - H0 * W0), (0, 5)))
    w2c = w2k[:, :8, :].reshape(KS, KS, 8, CP).reshape(KS, 40, CP)
    wf1 = w_fc1.reshape(32, CP, CP)[:25].reshape(25 * CP, CP)

    feats = pl.pallas_call(
        _conv_kernel,
        out_shape=jax.ShapeDtypeStruct((B, 25 * CP), jnp.float32),
        grid=(B // G,),
        in_specs=[
            pl.BlockSpec((G, SX, 8), lambda i: (i, 0, 0)),
            pl.BlockSpec((KS, 40, CP), lambda i: (0, 0, 0)),
            pl.BlockSpec((1, CP), lambda i: (0, 0)),
            pl.BlockSpec((KS, 40, CP), lambda i: (0, 0, 0)),
            pl.BlockSpec((1, CP), lambda i: (0, 0)),
        ],
        out_specs=pl.BlockSpec((G, 25 * CP), lambda i: (i, 0)),
        compiler_params=pltpu.CompilerParams(
            dimension_semantics=("parallel",),
            vmem_limit_bytes=64 * 1024 * 1024),
    )(x, w1k, b1, w2c, b2)

    fm = min(FM, B)
    out = pl.pallas_call(
        _fc_kernel,
        out_shape=jax.ShapeDtypeStruct((B, CP), jnp.float32),
        grid=(B // fm,),
        in_specs=[
            pl.BlockSpec((fm, 25 * CP), lambda i: (i, 0)),
            pl.BlockSpec((25 * CP, CP), lambda i: (0, 0)),
            pl.BlockSpec((1, CP), lambda i: (0, 0)),
            pl.BlockSpec((CP, CP), lambda i: (0, 0)),
            pl.BlockSpec((1, CP), lambda i: (0, 0)),
            pl.BlockSpec((CP, CP), lambda i: (0, 0)),
            pl.BlockSpec((1, CP), lambda i: (0, 0)),
        ],
        out_specs=pl.BlockSpec((fm, CP), lambda i: (i, 0)),
        compiler_params=pltpu.CompilerParams(
            dimension_semantics=("parallel",),
            vmem_limit_bytes=64 * 1024 * 1024),
    )(feats, wf1, b_fc1, w_fc2, b_fc2, w_fc3, b_fc3)
    return out[:, :NOUT]


def kernel(x_nchw, w1k, b1, w2k, b2, S1, S2,
           w_fc1, b_fc1, w_fc2, b_fc2, w_fc3, b_fc3):
    del S1, S2
    return _forward(x_nchw, w1k, b1, w2k, b2,
                    w_fc1, b_fc1, w_fc2, b_fc2, w_fc3, b_fc3)


# speedup vs baseline: 15.7697x; 13.3191x over previous
import jax
import jax.numpy as jnp
from jax.experimental import pallas as pl
from jax.experimental.pallas import tpu as pltpu

BL = 128
KS = 5
O_W1, O_B1, O_W2, O_B2 = 0, 450, 456, 2856


def _net_kernel(ws_ref, x_ref, w1l_ref, b1l_ref, w2l_ref, b2l_ref,
                w3l_ref, b3l_ref, o_ref, m1_ref, f3_ref):
    f32 = jnp.float32

    taps1 = [x_ref[ci, 32 * kh + kw:32 * kh + kw + 904, :]
             for ci in range(3) for kh in range(KS) for kw in range(KS)]

    def conv1_body(co, _):
        acc = jnp.zeros((904, BL), f32)
        for i in range(75):
            acc = acc + taps1[i] * ws_ref[O_W1 + co * 75 + i]
        r1 = jnp.maximum(acc + ws_ref[O_B1 + co], 0.0)
        e = r1[0:864].reshape(432, 2, BL)
        f = r1[32:896].reshape(432, 2, BL)
        m1 = jnp.maximum(jnp.maximum(e[:, 0], e[:, 1]),
                         jnp.maximum(f[:, 0], f[:, 1]))
        m1c = jnp.concatenate([m1, jnp.zeros((16, BL), f32)], axis=0)
        m1c = m1c.reshape(14, 32, BL)[:, :14].reshape(196, BL)
        m1_ref[co] = jnp.concatenate([m1c, jnp.zeros((12, BL), f32)], axis=0)
        return 0

    jax.lax.fori_loop(0, 6, conv1_body, 0)

    taps2 = [m1_ref[ci, 14 * kh + kw:14 * kh + kw + 144, :]
             for ci in range(6) for kh in range(KS) for kw in range(KS)]

    def conv2_body(co, _):
        acc = jnp.zeros((144, BL), f32)
        for i in range(150):
            acc = acc + taps2[i] * ws_ref[O_W2 + co * 150 + i]
        r2 = jnp.maximum(acc + ws_ref[O_B2 + co], 0.0)
        e2 = r2[0:128].reshape(64, 2, BL)
        f2 = r2[14:142].reshape(64, 2, BL)
        m2 = jnp.maximum(jnp.maximum(e2[:, 0], e2[:, 1]),
                         jnp.maximum(f2[:, 0], f2[:, 1]))
        m2 = jnp.concatenate([m2, jnp.zeros((6, BL), f32)], axis=0)
        f3_ref[co] = m2.reshape(5, 14, BL)[:, :5].reshape(25, BL)
        return 0

    jax.lax.fori_loop(0, 16, conv2_body, 0)

    feats = f3_ref[...].reshape(400, BL)
    h = jnp.maximum(jnp.dot(w1l_ref[...], feats,
                            preferred_element_type=f32) + b1l_ref[...], 0.0)
    h = jnp.maximum(jnp.dot(w2l_ref[...], h,
                            preferred_element_type=f32) + b2l_ref[...], 0.0)
    o_ref[...] = jnp.dot(w3l_ref[...], h,
                         preferred_element_type=f32) + b3l_ref[...]


@jax.jit
def _forward(x_nchw, w1k, b1, w2k, b2, w_fc1, b_fc1, w_fc2, b_fc2, w_fc3, b_fc3):
    B = x_nchw.shape[0]
    f32 = jnp.float32
    Bp = ((B + BL - 1) // BL) * BL
    xt = jnp.transpose(x_nchw, (1, 2, 3, 0)).reshape(3, 1024, B)
    xt = jnp.pad(xt, ((0, 0), (0, 16), (0, Bp - B)))

    w1s = jnp.transpose(w1k.reshape(KS, KS, 8, 128)[:, :, :3, :6],
                        (3, 2, 0, 1)).reshape(450)
    w2s = jnp.transpose(w2k[:, :6, :16], (2, 1, 0)).reshape(2400)
    ws = jnp.concatenate([w1s, b1[0, :6], w2s, b2[0, :16]])

    w1l = jnp.transpose(w_fc1.reshape(32, 128, 128)[:25, :16],
                        (2, 1, 0)).reshape(128, 400)
    b1l = jnp.broadcast_to(b_fc1[0][:, None], (128, BL))
    w2l = w_fc2.T
    b2l = jnp.broadcast_to(b_fc2[0][:, None], (128, BL))
    w3l = w_fc3.T
    b3l = jnp.broadcast_to(b_fc3[0][:, None], (128, BL))

    out = pl.pallas_call(
        _net_kernel,
        out_shape=jax.ShapeDtypeStruct((128, Bp), f32),
        grid=(Bp // BL,),
        in_specs=[
            pl.BlockSpec(memory_space=pltpu.SMEM),
            pl.BlockSpec((3, 1040, BL), lambda i: (0, 0, i)),
            pl.BlockSpec((128, 400), lambda i: (0, 0)),
            pl.BlockSpec((128, BL), lambda i: (0, 0)),
            pl.BlockSpec((128, 128), lambda i: (0, 0)),
            pl.BlockSpec((128, BL), lambda i: (0, 0)),
            pl.BlockSpec((128, 128), lambda i: (0, 0)),
            pl.BlockSpec((128, BL), lambda i: (0, 0)),
        ],
        out_specs=pl.BlockSpec((128, BL), lambda i: (0, i)),
        scratch_shapes=[
            pltpu.VMEM((6, 208, BL), f32),
            pltpu.VMEM((16, 25, BL), f32),
        ],
        compiler_params=pltpu.CompilerParams(
            dimension_semantics=("parallel",),
            vmem_limit_bytes=64 * 1024 * 1024),
    )(ws, xt, w1l, b1l, w2l, b2l, w3l, b3l)
    return out[:10, :B].T


def kernel(x_nchw, w1k, b1, w2k, b2, S1, S2,
           w_fc1, b_fc1, w_fc2, b_fc2, w_fc3, b_fc3):
    del S1, S2
    return _forward(x_nchw, w1k, b1, w2k, b2,
                    w_fc1, b_fc1, w_fc2, b_fc2, w_fc3, b_fc3)
